# baseline (speedup 1.0000x reference)
import jax
import jax.numpy as jnp
from jax import lax
from jax.experimental import pallas as pl
from jax.experimental.pallas import tpu as pltpu
from jax.experimental.pallas import tpu_sc as plsc

NC = 2
NS = 16
NW = NC * NS
CN = 16
ROWS = CN * 7
GC = 2
NIB = 4
FEAT = 128
INV7 = float(1.0 / 7.0)


def _tec_body(x_hbm, idx_hbm, out_hbm,
              idx0, idx1, idx2, idx3, rows_a, rows_b, out0, out1,
              si0, si1, si2, si3, sa0, sa1, sb0, sb1, so0, so1):
    wid = lax.axis_index("s") * NC + lax.axis_index("c")
    npw = out_hbm.shape[0] // NW
    nchunk = npw // CN
    nsuper = nchunk // GC
    chunk_base = wid * nchunk

    lane = lax.iota(jnp.int32, 16)
    lane7 = lane * 7

    idx_bufs = (idx0, idx1, idx2, idx3)
    idx_sems = (si0, si1, si2, si3)
    rows_bufs = (rows_a, rows_b)
    gat_sems = ((sa0, sa1), (sb0, sb1))
    out_bufs = (out0, out1)
    out_sems = (so0, so1)

    def idx_src(s):
        return idx_hbm.at[pl.ds(chunk_base + s * GC, GC)]

    def idx_fetch(s, m):
        pltpu.async_copy(idx_src(s), idx_bufs[m], idx_sems[m])

    def gather_fire(s, m, b):
        pltpu.make_async_copy(idx_src(s), idx_bufs[m], idx_sems[m]).wait()
        for j in range(GC):
            pltpu.async_copy(
                x_hbm.at[idx_bufs[m].at[j]], rows_bufs[b].at[j],
                gat_sems[b][j])

    def compute_group(s, b):
        for j in range(GC):
            pltpu.make_async_copy(
                x_hbm.at[idx_bufs[0].at[j]], rows_bufs[b].at[j],
                gat_sems[b][j]).wait()
            rows = rows_bufs[b].at[j]
            node0 = (chunk_base + s * GC + j) * CN
            out_dst = out_hbm.at[pl.ds(node0, CN)]

            @pl.when(s > 0)
            def _drain_prev():
                pltpu.make_async_copy(
                    out_bufs[j], out_dst, out_sems[j]).wait()

            @plsc.parallel_loop(0, FEAT, unroll=4)
            def _pool(f):
                p0 = 7 * f
                vs = []
                for k in range(7):
                    p = p0 + k
                    rvec = lane7 + (p >> 7)
                    cvec = jnp.full((16,), p & 127, jnp.int32)
                    vs.append(plsc.load_gather(rows, [rvec, cvec]))
                acc = ((vs[0] + vs[1]) + (vs[2] + vs[3])) + (
                    (vs[4] + vs[5]) + vs[6])
                fvec = jnp.full((16,), f, jnp.int32)
                plsc.store_scatter(out_bufs[j], [lane, fvec], acc * INV7)

            pltpu.async_copy(out_bufs[j], out_dst, out_sems[j])

    idx_fetch(0, 0)
    idx_fetch(1, 1)
    gather_fire(0, 0, 0)
    idx_fetch(2, 2)
    gather_fire(1, 1, 1)
    idx_fetch(3, 3)

    def loop_body(i, carry):
        for b in range(NIB):
            s = i * NIB + b

            @pl.when(s < nsuper)
            def _do():
                compute_group(s, b % 2)

                @pl.when(s + 2 < nsuper)
                def _prefetch():
                    gather_fire(s + 2, (b + 2) % NIB, b % 2)

                    @pl.when(s + 4 < nsuper)
                    def _idx():
                        idx_fetch(s + 4, b)

        return carry

    lax.fori_loop(0, (nsuper + NIB - 1) // NIB, loop_body, 0)

    last0 = (chunk_base + (nsuper - 1) * GC) * CN
    for j in range(GC):
        pltpu.make_async_copy(
            out_bufs[j], out_hbm.at[pl.ds(last0 + j * CN, CN)],
            out_sems[j]).wait()


def _build(n_pad):
    mesh = plsc.VectorSubcoreMesh(core_axis_name="c", subcore_axis_name="s")
    return pl.kernel(
        _tec_body,
        mesh=mesh,
        out_type=jax.ShapeDtypeStruct((n_pad, FEAT), jnp.float32),
        scratch_types=[
            pltpu.VMEM((GC, ROWS), jnp.int32),
            pltpu.VMEM((GC, ROWS), jnp.int32),
            pltpu.VMEM((GC, ROWS), jnp.int32),
            pltpu.VMEM((GC, ROWS), jnp.int32),
            pltpu.VMEM((GC, ROWS, FEAT), jnp.float32),
            pltpu.VMEM((GC, ROWS, FEAT), jnp.float32),
            pltpu.VMEM((CN, FEAT), jnp.float32),
            pltpu.VMEM((CN, FEAT), jnp.float32),
            pltpu.SemaphoreType.DMA,
            pltpu.SemaphoreType.DMA,
            pltpu.SemaphoreType.DMA,
            pltpu.SemaphoreType.DMA,
            pltpu.SemaphoreType.DMA,
            pltpu.SemaphoreType.DMA,
            pltpu.SemaphoreType.DMA,
            pltpu.SemaphoreType.DMA,
            pltpu.SemaphoreType.DMA,
            pltpu.SemaphoreType.DMA,
        ],
        compiler_params=pltpu.CompilerParams(needs_layout_passes=False),
    )


@jax.jit
def kernel(x, hex):
    n = hex.shape[0]
    group_stride = NW * CN * GC
    n_pad = -(-n // group_stride) * group_stride
    idx = hex.reshape(-1)
    idx = jnp.pad(idx, (0, n_pad * 7 - idx.shape[0]))
    idx = idx.reshape(n_pad // CN, ROWS)
    out = _build(n_pad)(x, idx)
    return out[:n]
